# baseline (speedup 1.0000x reference)
import functools

import jax
import jax.numpy as jnp
from jax import lax
from jax.experimental import pallas as pl
from jax.experimental.pallas import tpu as pltpu
from jax.experimental.pallas import tpu_sc as plsc

E = 8
K = 2
D = 768
T = 32768
LANES = 16
NUM_CORES = 2
NUM_SUBCORES = 16
NW = NUM_CORES * NUM_SUBCORES
TM = 4096
CH = 2
TCH = T // CH
TPW = TCH // NW


def _logits_body(w_ref, h_ref, out_ref, flat_ref):
    t = lax.dot_general(
        w_ref[...], h_ref[...], (((1,), (1,)), ((), ())),
        preferred_element_type=jnp.float32,
        precision=lax.Precision.DEFAULT,
    )
    out_ref[...] = t
    flat_ref[...] = jnp.concatenate(
        [t[:, 128 * a:128 * (a + 1)].reshape(E * 128) for a in range(TM // 128)]
    )


def _logits_tc(h, W, j):
    off = j * (TCH // TM)
    return pl.pallas_call(
        _logits_body,
        grid=(TCH // TM,),
        in_specs=[
            pl.BlockSpec((E, D), lambda i: (0, 0)),
            pl.BlockSpec((TM, D), lambda i, off=off: (i + off, 0)),
        ],
        out_specs=[
            pl.BlockSpec((E, TM), lambda i: (0, i)),
            pl.BlockSpec((TM * E,), lambda i: (i,)),
        ],
        out_shape=[
            jax.ShapeDtypeStruct((E, TCH), jnp.float32),
            jax.ShapeDtypeStruct((TCH * E,), jnp.float32),
        ],
    )(W, h)


def _route_body(logits_hbm, w_hbm, i_hbm, lg_v, w_v, i_v, sem):
    wid = lax.axis_index("c") * NUM_SUBCORES + lax.axis_index("s")
    base = wid * TPW
    pltpu.async_copy(
        logits_hbm.at[pl.ds(base * E, TPW * E)], lg_v, sem).wait()

    iota = lax.broadcasted_iota(jnp.int32, (LANES,), 0)
    zeros = jnp.zeros((LANES,), jnp.int32)
    ones = jnp.full((LANES,), 1, jnp.int32)

    @pl.loop(0, TPW, step=LANES)
    def _chunk(t0):
        blk = (t0 // 128) * (128 * E) + (t0 % 128)
        l0 = lg_v[pl.ds(blk, LANES)]
        l1 = lg_v[pl.ds(blk + 128, LANES)]
        gt = l1 > l0
        best = jnp.where(gt, l1, l0)
        bidx = jnp.where(gt, ones, zeros)
        second = jnp.where(gt, l0, l1)
        sidx = jnp.where(gt, zeros, ones)
        for e in range(2, E):
            ev = jnp.full((LANES,), e, jnp.int32)
            le = lg_v[pl.ds(blk + 128 * e, LANES)]
            gt_b = le > best
            gt_s = le > second
            second = jnp.where(gt_b, best, jnp.where(gt_s, le, second))
            sidx = jnp.where(gt_b, bidx, jnp.where(gt_s, ev, sidx))
            best = jnp.where(gt_b, le, best)
            bidx = jnp.where(gt_b, ev, bidx)
        r = jnp.exp(second - best)
        denom = r + 1.0
        w_v[pl.ds(t0, LANES)] = 1.0 / denom
        w_v[pl.ds(TPW + t0, LANES)] = r / denom
        i_v[pl.ds(t0, LANES)] = bidx
        i_v[pl.ds(TPW + t0, LANES)] = sidx

    outs = [
        pltpu.async_copy(w_v.at[pl.ds(0, TPW)],
                         w_hbm.at[pl.ds(base, TPW)], sem),
        pltpu.async_copy(w_v.at[pl.ds(TPW, TPW)],
                         w_hbm.at[pl.ds(TCH + base, TPW)], sem),
        pltpu.async_copy(i_v.at[pl.ds(0, TPW)],
                         i_hbm.at[pl.ds(base, TPW)], sem),
        pltpu.async_copy(i_v.at[pl.ds(TPW, TPW)],
                         i_hbm.at[pl.ds(TCH + base, TPW)], sem),
    ]
    for c in outs:
        c.wait()


@functools.cache
def _route_sc():
    return pl.kernel(
        _route_body,
        out_type=[
            jax.ShapeDtypeStruct((K * TCH,), jnp.float32),
            jax.ShapeDtypeStruct((K * TCH,), jnp.int32),
        ],
        mesh=plsc.VectorSubcoreMesh(
            core_axis_name="c", subcore_axis_name="s",
            num_cores=NUM_CORES, num_subcores=NUM_SUBCORES,
        ),
        scratch_types=[
            pltpu.VMEM((E * TPW,), jnp.float32),
            pltpu.VMEM((K * TPW,), jnp.float32),
            pltpu.VMEM((K * TPW,), jnp.int32),
            pltpu.SemaphoreType.DMA,
        ],
        compiler_params=pltpu.CompilerParams(needs_layout_passes=False),
    )


def kernel(hidden_states, W):
    parts = [_logits_tc(hidden_states, W, j) for j in range(CH)]
    routed = [_route_sc()(fl) for (_, fl) in parts]
    logits = jnp.concatenate([lt.T for (lt, _) in parts], axis=0)
    w = jnp.concatenate(
        [wf.reshape(K, TCH) for (wf, _) in routed], axis=1).T
    i = jnp.concatenate(
        [if_.reshape(K, TCH) for (_, if_) in routed], axis=1).T
    return logits, w, i


# speedup vs baseline: 1.0587x; 1.0587x over previous
import functools

import jax
import jax.numpy as jnp
from jax import lax
from jax.experimental import pallas as pl
from jax.experimental.pallas import tpu as pltpu
from jax.experimental.pallas import tpu_sc as plsc

E = 8
K = 2
D = 768
T = 32768
LANES = 16
NUM_CORES = 2
NUM_SUBCORES = 16
NW = NUM_CORES * NUM_SUBCORES
TM = 4096
TCH = T
TPW = TCH // NW


def _logits_body(w_ref, h_ref, out_ref, flat_ref):
    t = lax.dot_general(
        w_ref[...], h_ref[...], (((1,), (1,)), ((), ())),
        preferred_element_type=jnp.float32,
        precision=lax.Precision.DEFAULT,
    )
    out_ref[...] = t
    flat_ref[...] = jnp.concatenate(
        [t[:, 128 * a:128 * (a + 1)].reshape(E * 128) for a in range(TM // 128)]
    )


def _logits_tc(h, W):
    return pl.pallas_call(
        _logits_body,
        grid=(TCH // TM,),
        in_specs=[
            pl.BlockSpec((E, D), lambda i: (0, 0)),
            pl.BlockSpec((TM, D), lambda i: (i, 0)),
        ],
        out_specs=[
            pl.BlockSpec((E, TM), lambda i: (0, i)),
            pl.BlockSpec((TM * E,), lambda i: (i,)),
        ],
        out_shape=[
            jax.ShapeDtypeStruct((E, TCH), jnp.float32),
            jax.ShapeDtypeStruct((TCH * E,), jnp.float32),
        ],
    )(W, h)


def _route_body(logits_hbm, w_hbm, i_hbm, lg_v, w_v, i_v, sem):
    wid = lax.axis_index("c") * NUM_SUBCORES + lax.axis_index("s")
    base = wid * TPW
    pltpu.async_copy(
        logits_hbm.at[pl.ds(base * E, TPW * E)], lg_v, sem).wait()

    iota = lax.broadcasted_iota(jnp.int32, (LANES,), 0)
    zeros = jnp.zeros((LANES,), jnp.int32)
    ones = jnp.full((LANES,), 1, jnp.int32)

    @pl.loop(0, TPW, step=LANES)
    def _chunk(t0):
        blk = (t0 // 128) * (128 * E) + (t0 % 128)
        l0 = lg_v[pl.ds(blk, LANES)]
        l1 = lg_v[pl.ds(blk + 128, LANES)]
        gt = l1 > l0
        best = jnp.where(gt, l1, l0)
        bidx = jnp.where(gt, ones, zeros)
        second = jnp.where(gt, l0, l1)
        sidx = jnp.where(gt, zeros, ones)
        for e in range(2, E):
            ev = jnp.full((LANES,), e, jnp.int32)
            le = lg_v[pl.ds(blk + 128 * e, LANES)]
            gt_b = le > best
            gt_s = le > second
            second = jnp.where(gt_b, best, jnp.where(gt_s, le, second))
            sidx = jnp.where(gt_b, bidx, jnp.where(gt_s, ev, sidx))
            best = jnp.where(gt_b, le, best)
            bidx = jnp.where(gt_b, ev, bidx)
        r = jnp.exp(second - best)
        denom = r + 1.0
        w_v[pl.ds(t0, LANES)] = 1.0 / denom
        w_v[pl.ds(TPW + t0, LANES)] = r / denom
        i_v[pl.ds(t0, LANES)] = bidx
        i_v[pl.ds(TPW + t0, LANES)] = sidx

    outs = [
        pltpu.async_copy(w_v.at[pl.ds(0, TPW)],
                         w_hbm.at[pl.ds(base, TPW)], sem),
        pltpu.async_copy(w_v.at[pl.ds(TPW, TPW)],
                         w_hbm.at[pl.ds(TCH + base, TPW)], sem),
        pltpu.async_copy(i_v.at[pl.ds(0, TPW)],
                         i_hbm.at[pl.ds(base, TPW)], sem),
        pltpu.async_copy(i_v.at[pl.ds(TPW, TPW)],
                         i_hbm.at[pl.ds(TCH + base, TPW)], sem),
    ]
    for c in outs:
        c.wait()


@functools.cache
def _route_sc():
    return pl.kernel(
        _route_body,
        out_type=[
            jax.ShapeDtypeStruct((K * TCH,), jnp.float32),
            jax.ShapeDtypeStruct((K * TCH,), jnp.int32),
        ],
        mesh=plsc.VectorSubcoreMesh(
            core_axis_name="c", subcore_axis_name="s",
            num_cores=NUM_CORES, num_subcores=NUM_SUBCORES,
        ),
        scratch_types=[
            pltpu.VMEM((E * TPW,), jnp.float32),
            pltpu.VMEM((K * TPW,), jnp.float32),
            pltpu.VMEM((K * TPW,), jnp.int32),
            pltpu.SemaphoreType.DMA,
        ],
        compiler_params=pltpu.CompilerParams(needs_layout_passes=False),
    )


def kernel(hidden_states, W):
    warm_w, warm_i = _route_sc()(jnp.zeros((TCH * E,), jnp.float32))
    logits_t, lg_flat = _logits_tc(hidden_states, W)
    w_flat, i_flat = _route_sc()(lg_flat)
    w_flat, i_flat = jax.lax.optimization_barrier(
        (w_flat, i_flat, warm_w, warm_i))[:2]
    return (
        logits_t.T,
        w_flat.reshape(K, T).T,
        i_flat.reshape(K, T).T,
    )


# speedup vs baseline: 1.1345x; 1.0716x over previous
import functools

import jax
import jax.numpy as jnp
from jax import lax
from jax.experimental import pallas as pl
from jax.experimental.pallas import tpu as pltpu
from jax.experimental.pallas import tpu_sc as plsc

E = 8
K = 2
D = 768
T = 32768
LANES = 16
NUM_CORES = 2
NUM_SUBCORES = 16
NW = NUM_CORES * NUM_SUBCORES
TM = 4096
TCH = T
TPW = TCH // NW


def _logits_body(w_ref, h_ref, out_ref, flat_ref):
    t = lax.dot_general(
        w_ref[...], h_ref[...], (((1,), (1,)), ((), ())),
        preferred_element_type=jnp.float32,
        precision=lax.Precision.DEFAULT,
    )
    out_ref[...] = t
    flat_ref[...] = jnp.concatenate(
        [t[:, 128 * a:128 * (a + 1)].reshape(E * 128) for a in range(TM // 128)]
    )


def _logits_tc(h, W):
    return pl.pallas_call(
        _logits_body,
        grid=(TCH // TM,),
        in_specs=[
            pl.BlockSpec((E, D), lambda i: (0, 0)),
            pl.BlockSpec((TM, D), lambda i: (i, 0)),
        ],
        out_specs=[
            pl.BlockSpec((E, TM), lambda i: (0, i)),
            pl.BlockSpec((TM * E,), lambda i: (i,)),
        ],
        out_shape=[
            jax.ShapeDtypeStruct((E, TCH), jnp.float32),
            jax.ShapeDtypeStruct((TCH * E,), jnp.float32),
        ],
    )(W, h)


def _route_body(logits_hbm, w_hbm, i_hbm, lg_v, w_v, i_v, sem):
    wid = lax.axis_index("c") * NUM_SUBCORES + lax.axis_index("s")
    base = wid * TPW
    pltpu.async_copy(
        logits_hbm.at[pl.ds(base * E, TPW * E)], lg_v, sem).wait()

    iota = lax.broadcasted_iota(jnp.int32, (LANES,), 0)
    zeros = jnp.zeros((LANES,), jnp.int32)
    ones = jnp.full((LANES,), 1, jnp.int32)

    @pl.loop(0, TPW, step=LANES)
    def _chunk(t0):
        blk = (t0 // 128) * (128 * E) + (t0 % 128)
        l0 = lg_v[pl.ds(blk, LANES)]
        l1 = lg_v[pl.ds(blk + 128, LANES)]
        gt = l1 > l0
        best = jnp.where(gt, l1, l0)
        bidx = jnp.where(gt, ones, zeros)
        second = jnp.where(gt, l0, l1)
        sidx = jnp.where(gt, zeros, ones)
        for e in range(2, E):
            ev = jnp.full((LANES,), e, jnp.int32)
            le = lg_v[pl.ds(blk + 128 * e, LANES)]
            gt_b = le > best
            gt_s = le > second
            second = jnp.where(gt_b, best, jnp.where(gt_s, le, second))
            sidx = jnp.where(gt_b, bidx, jnp.where(gt_s, ev, sidx))
            best = jnp.where(gt_b, le, best)
            bidx = jnp.where(gt_b, ev, bidx)
        r = jnp.exp(second - best)
        denom = r + 1.0
        pos = (t0 // 128) * 256 + (t0 % 128)
        w_v[pl.ds(pos, LANES)] = 1.0 / denom
        w_v[pl.ds(pos + 128, LANES)] = r / denom
        i_v[pl.ds(pos, LANES)] = bidx
        i_v[pl.ds(pos + 128, LANES)] = sidx

    outs = [
        pltpu.async_copy(w_v, w_hbm.at[pl.ds(base * K, TPW * K)], sem),
        pltpu.async_copy(i_v, i_hbm.at[pl.ds(base * K, TPW * K)], sem),
    ]
    for c in outs:
        c.wait()


@functools.cache
def _route_sc():
    return pl.kernel(
        _route_body,
        out_type=[
            jax.ShapeDtypeStruct((K * TCH,), jnp.float32),
            jax.ShapeDtypeStruct((K * TCH,), jnp.int32),
        ],
        mesh=plsc.VectorSubcoreMesh(
            core_axis_name="c", subcore_axis_name="s",
            num_cores=NUM_CORES, num_subcores=NUM_SUBCORES,
        ),
        scratch_types=[
            pltpu.VMEM((E * TPW,), jnp.float32),
            pltpu.VMEM((K * TPW,), jnp.float32),
            pltpu.VMEM((K * TPW,), jnp.int32),
            pltpu.SemaphoreType.DMA,
        ],
        compiler_params=pltpu.CompilerParams(needs_layout_passes=False),
    )


def _untangle(flat):
    return flat.reshape(T // 128, K, 128).transpose(0, 2, 1).reshape(T, K)


def kernel(hidden_states, W):
    logits_t, lg_flat = _logits_tc(hidden_states, W)
    w_flat, i_flat = _route_sc()(lg_flat)
    return (
        logits_t.T,
        _untangle(w_flat),
        _untangle(i_flat),
    )
